# baseline (speedup 1.0000x reference)
import functools

import jax
import jax.numpy as jnp
from jax import lax
from jax.experimental import pallas as pl
from jax.experimental.pallas import tpu as pltpu
from jax.experimental.pallas import tpu_sc as plsc

_NC = 2
_NS = 16
_NW = _NC * _NS
_LANES = 16
_NBUF = 4


@functools.partial(jax.jit, static_argnames=("n_chunks", "chunk", "t", "d"))
def _sc_embed(ids_3d, token_table, pos_table, *, n_chunks, chunk, t, d):
    n_rows = _NW * n_chunks * chunk
    rows_per_w = n_chunks * chunk
    n_rounds = n_chunks // _NBUF
    n_groups = chunk // t
    slabs = d // _LANES
    mesh = plsc.VectorSubcoreMesh(core_axis_name="c", subcore_axis_name="s")

    sup = 2 * chunk
    n_sup = n_chunks // 2

    @functools.partial(
        pl.kernel,
        out_type=jax.ShapeDtypeStruct((n_rows, d), jnp.float32),
        mesh=mesh,
        scratch_types=[
            pltpu.VMEM((n_chunks, chunk), jnp.int32),
            pltpu.VMEM((3, sup, d), jnp.float32),
            pltpu.VMEM((t, d), jnp.float32),
        ]
        + [pltpu.SemaphoreType.DMA] * 6,
    )
    def body(ids_hbm, table_hbm, pos_hbm, out_hbm, idx_v, rows, pos_v, *sems):
        sgs, sos = sems[:3], sems[3:]
        wid = lax.axis_index("s") * _NC + lax.axis_index("c")
        base = wid * rows_per_w
        pltpu.sync_copy(pos_hbm, pos_v)
        pltpu.sync_copy(ids_hbm.at[wid], idx_v)

        def ga(s, b, h):
            return pltpu.make_async_copy(
                table_hbm.at[idx_v.at[2 * s + h]],
                rows.at[b, pl.ds(h * chunk, chunk)], sgs[b])

        def wb(s, b):
            return pltpu.make_async_copy(
                rows.at[b], out_hbm.at[pl.ds(base + s * sup, sup)], sos[b])

        def add_sup(b):
            for j in range(slabs):
                sl = pl.ds(j * _LANES, _LANES)
                pvs = tuple(pos_v[k, sl] for k in range(t))

                def grp(g, pvs):
                    r0 = g * t
                    for k in range(t):
                        rows[b, r0 + k, sl] = rows[b, r0 + k, sl] + pvs[k]
                    return pvs

                lax.fori_loop(0, sup // t, grp, pvs)

        def step(s, b, drain, ahead):
            ga(s, b, 0).wait()
            ga(s, b, 1).wait()
            if drain:
                wb(s - 2, (b + 1) % 3).wait()
            if ahead:
                ga(s + 1, (b + 1) % 3, 0).start()
                ga(s + 1, (b + 1) % 3, 1).start()
            add_sup(b)
            wb(s, b).start()

        ga(0, 0, 0).start()
        ga(0, 0, 1).start()
        step(0, 0, False, True)
        step(1, 1, False, True)

        def mid(p, _):
            s0 = 3 * p + 2
            step(s0, 2, True, True)
            step(s0 + 1, 0, True, True)
            step(s0 + 2, 1, True, True)
            return 0

        lax.fori_loop(0, (n_sup - 4) // 3, mid, 0)

        step(n_sup - 2, 2, True, True)
        step(n_sup - 1, 0, True, False)
        wb(n_sup - 2, 2).wait()
        wb(n_sup - 1, 0).wait()

    return body(ids_3d, token_table, pos_table)


def kernel(input_ids, token_table, pos_table):
    bq, tq = input_ids.shape
    vocab, d = token_table.shape
    n = bq * tq
    chunk = 128
    assert n % (_NW * chunk) == 0 and chunk % tq == 0 and d % _LANES == 0
    n_chunks = n // (_NW * chunk)
    assert n_chunks % 2 == 0 and (n_chunks // 2 - 4) % 3 == 0
    ids_3d = input_ids.astype(jnp.int32).reshape(_NW, n_chunks, chunk)
    out = _sc_embed(ids_3d, token_table, pos_table,
                    n_chunks=n_chunks, chunk=chunk, t=tq, d=d)
    return out.reshape(bq, tq, d)


# speedup vs baseline: 1.0054x; 1.0054x over previous
import functools

import jax
import jax.numpy as jnp
from jax import lax
from jax.experimental import pallas as pl
from jax.experimental.pallas import tpu as pltpu
from jax.experimental.pallas import tpu_sc as plsc

_NC = 2
_NS = 16
_NW = _NC * _NS
_LANES = 16
_NBUF = 4


@functools.partial(jax.jit, static_argnames=("n_chunks", "chunk", "t", "d"))
def _sc_embed(ids_3d, token_table, pos_table, *, n_chunks, chunk, t, d):
    n_rows = _NW * n_chunks * chunk
    rows_per_w = n_chunks * chunk
    n_rounds = n_chunks // _NBUF
    n_groups = chunk // t
    slabs = d // _LANES
    mesh = plsc.VectorSubcoreMesh(core_axis_name="c", subcore_axis_name="s")

    @functools.partial(
        pl.kernel,
        out_type=jax.ShapeDtypeStruct((n_rows, d), jnp.float32),
        mesh=mesh,
        scratch_types=[
            pltpu.VMEM((n_chunks, chunk), jnp.int32),
            pltpu.VMEM((_NBUF, chunk, d), jnp.float32),
            pltpu.VMEM((2, chunk, d), jnp.float32),
            pltpu.VMEM((t, d), jnp.float32),
        ]
        + [pltpu.SemaphoreType.DMA] * (_NBUF + 2),
    )
    def body(ids_hbm, table_hbm, pos_hbm, out_hbm, idx_v, rows, wrows, pos_v,
             *sems):
        sgs, sos = sems[:_NBUF], sems[_NBUF:]
        wid = lax.axis_index("s") * _NC + lax.axis_index("c")
        base = wid * rows_per_w
        pltpu.sync_copy(pos_hbm, pos_v)
        pltpu.sync_copy(ids_hbm.at[wid], idx_v)

        def ga(c, b):
            return pltpu.make_async_copy(
                table_hbm.at[idx_v.at[c]], rows.at[b], sgs[b])

        def wb(c, w):
            return pltpu.make_async_copy(
                wrows.at[w], out_hbm.at[pl.ds(base + c * chunk, chunk)],
                sos[w])

        def add_chunk(b, w):
            for j in range(slabs):
                sl = pl.ds(j * _LANES, _LANES)
                pvs = tuple(pos_v[k, sl] for k in range(t))

                def grp(g, pvs):
                    r0 = g * t
                    for k in range(t):
                        wrows[w, r0 + k, sl] = rows[b, r0 + k, sl] + pvs[k]
                    return pvs

                lax.fori_loop(0, n_groups, grp, pvs)

        def step(c, k, drain, ahead):
            ga(c, k).wait()
            if drain:
                wb(c - 2, k % 2).wait()
            if ahead:
                ga(c + 3, (k + 3) % _NBUF).start()
            add_chunk(k, k % 2)
            wb(c, k % 2).start()

        ga(0, 0).start()
        ga(1, 1).start()
        ga(2, 2).start()

        for k in range(_NBUF):
            step(k, k, drain=k >= 2, ahead=True)

        def mid(p, _):
            c0 = p * _NBUF
            for k in range(_NBUF):
                step(c0 + k, k, drain=True, ahead=True)
            return 0

        lax.fori_loop(1, n_rounds - 1, mid, 0)

        cL = (n_rounds - 1) * _NBUF
        for k in range(_NBUF):
            step(cL + k, k, drain=True, ahead=k < 1)

        wb(cL + 2, 0).wait()
        wb(cL + 3, 1).wait()

    return body(ids_3d, token_table, pos_table)


def kernel(input_ids, token_table, pos_table):
    bq, tq = input_ids.shape
    vocab, d = token_table.shape
    n = bq * tq
    chunk = 128
    assert n % (_NW * chunk) == 0 and chunk % tq == 0 and d % _LANES == 0
    n_chunks = n // (_NW * chunk)
    assert n_chunks % _NBUF == 0 and n_chunks // _NBUF >= 2
    ids_3d = input_ids.astype(jnp.int32).reshape(_NW, n_chunks, chunk)
    out = _sc_embed(ids_3d, token_table, pos_table,
                    n_chunks=n_chunks, chunk=chunk, t=tq, d=d)
    return out.reshape(bq, tq, d)


# speedup vs baseline: 1.0196x; 1.0141x over previous
import functools

import jax
import jax.numpy as jnp
from jax import lax
from jax.experimental import pallas as pl
from jax.experimental.pallas import tpu as pltpu
from jax.experimental.pallas import tpu_sc as plsc

_NC = 2
_NS = 16
_NW = _NC * _NS
_LANES = 16
_NBUF = 4


@functools.partial(jax.jit, static_argnames=("n_chunks", "chunk", "t", "d"))
def _sc_embed(ids_3d, token_table, pos_table, *, n_chunks, chunk, t, d):
    n_rows = _NW * n_chunks * chunk
    rows_per_w = n_chunks * chunk
    n_rounds = n_chunks // _NBUF
    n_groups = chunk // t
    slabs = d // _LANES
    mesh = plsc.VectorSubcoreMesh(core_axis_name="c", subcore_axis_name="s")

    @functools.partial(
        pl.kernel,
        out_type=jax.ShapeDtypeStruct((n_rows, d), jnp.float32),
        mesh=mesh,
        compiler_params=pltpu.CompilerParams(use_tc_tiling_on_sc=False),
        scratch_types=[
            pltpu.VMEM((n_chunks, chunk), jnp.int32),
            pltpu.VMEM((_NBUF, chunk, d), jnp.float32),
            pltpu.VMEM((t, d), jnp.float32),
        ]
        + [pltpu.SemaphoreType.DMA] * (2 * _NBUF),
    )
    def body(ids_hbm, table_hbm, pos_hbm, out_hbm, idx_v, rows, pos_v, *sems):
        sgs, sos = sems[:_NBUF], sems[_NBUF:]
        wid = lax.axis_index("s") * _NC + lax.axis_index("c")
        base = wid * rows_per_w
        pltpu.sync_copy(pos_hbm, pos_v)
        pltpu.sync_copy(ids_hbm.at[wid], idx_v)

        def ga(c, b):
            return pltpu.make_async_copy(
                table_hbm.at[idx_v.at[c]], rows.at[b], sgs[b])

        def wb(c, b):
            return pltpu.make_async_copy(
                rows.at[b], out_hbm.at[pl.ds(base + c * chunk, chunk)], sos[b])

        def add_chunk(b):
            for j in range(slabs):
                sl = pl.ds(j * _LANES, _LANES)
                pvs = tuple(pos_v[k, sl] for k in range(t))

                def grp(g, pvs):
                    r0 = g * t
                    for k in range(t):
                        rows[b, r0 + k, sl] = rows[b, r0 + k, sl] + pvs[k]
                    return pvs

                lax.fori_loop(0, n_groups, grp, pvs)

        ga(0, 0).start()
        ga(1, 1).start()

        for k in range(_NBUF):
            ga(k, k).wait()
            if k < 2:
                ga(k + 2, k + 2).start()
            else:
                wb(k - 2, k - 2).wait()
                ga(k + 2, k - 2).start()
            add_chunk(k)
            wb(k, k).start()

        def mid(p, _):
            c0 = p * _NBUF
            for k in range(_NBUF):
                c = c0 + k
                ga(c, k).wait()
                b2 = (k + 2) % _NBUF
                wb(c - 2, b2).wait()
                ga(c + 2, b2).start()
                add_chunk(k)
                wb(c, k).start()
            return 0

        lax.fori_loop(1, n_rounds - 1, mid, 0)

        cL = (n_rounds - 1) * _NBUF
        for k in range(_NBUF):
            c = cL + k
            ga(c, k).wait()
            if k < 2:
                b2 = (k + 2) % _NBUF
                wb(c - 2, b2).wait()
                ga(c + 2, b2).start()
            add_chunk(k)
            wb(c, k).start()

        for k in range(_NBUF):
            wb(cL + k, k).wait()

    return body(ids_3d, token_table, pos_table)


def kernel(input_ids, token_table, pos_table):
    bq, tq = input_ids.shape
    vocab, d = token_table.shape
    n = bq * tq
    chunk = 128
    assert n % (_NW * chunk) == 0 and chunk % tq == 0 and d % _LANES == 0
    n_chunks = n // (_NW * chunk)
    assert n_chunks % _NBUF == 0 and n_chunks // _NBUF >= 2
    ids_3d = input_ids.astype(jnp.int32).reshape(_NW, n_chunks, chunk)
    out = _sc_embed(ids_3d, token_table, pos_table,
                    n_chunks=n_chunks, chunk=chunk, t=tq, d=d)
    return out.reshape(bq, tq, d)


# speedup vs baseline: 1.0273x; 1.0076x over previous
import functools

import jax
import jax.numpy as jnp
from jax import lax
from jax.experimental import pallas as pl
from jax.experimental.pallas import tpu as pltpu
from jax.experimental.pallas import tpu_sc as plsc

_NC = 2
_NS = 16
_NW = _NC * _NS
_LANES = 16
_NBUF = 4


@functools.partial(jax.jit, static_argnames=("n_chunks", "chunk", "t", "d"))
def _sc_embed(ids_3d, token_table, pos_table, *, n_chunks, chunk, t, d):
    n_rows = _NW * n_chunks * chunk
    rows_per_w = n_chunks * chunk
    n_rounds = n_chunks // _NBUF
    n_groups = chunk // t
    slabs = d // _LANES
    mesh = plsc.VectorSubcoreMesh(core_axis_name="c", subcore_axis_name="s")

    @functools.partial(
        pl.kernel,
        out_type=jax.ShapeDtypeStruct((n_rows, d), jnp.float32),
        mesh=mesh,
        compiler_params=pltpu.CompilerParams(use_tc_tiling_on_sc=False),
        scratch_types=[
            pltpu.VMEM((n_chunks, chunk), jnp.int32),
            pltpu.VMEM((_NBUF, chunk, d), jnp.float32),
            pltpu.VMEM((t, d), jnp.float32),
        ]
        + [pltpu.SemaphoreType.DMA] * (2 * _NBUF + 2),
    )
    def body(ids_hbm, table_hbm, pos_hbm, out_hbm, idx_v, rows, pos_v, *sems):
        sgs, sos = sems[:_NBUF], sems[_NBUF : 2 * _NBUF]
        wid = lax.axis_index("s") * _NC + lax.axis_index("c")
        base = wid * rows_per_w
        idx_cp = pltpu.make_async_copy(ids_hbm.at[wid], idx_v, sems[-1])
        pos_cp = pltpu.make_async_copy(pos_hbm, pos_v, sems[-2])
        idx_cp.start()
        pos_cp.start()
        idx_cp.wait()

        def ga(c, b):
            return pltpu.make_async_copy(
                table_hbm.at[idx_v.at[c]], rows.at[b], sgs[b])

        def wb(c, b):
            return pltpu.make_async_copy(
                rows.at[b], out_hbm.at[pl.ds(base + c * chunk, chunk)], sos[b])

        def add_chunk(b):
            for j in range(slabs):
                sl = pl.ds(j * _LANES, _LANES)
                pvs = tuple(pos_v[k, sl] for k in range(t))

                def grp(g, pvs):
                    r0 = g * t
                    for k in range(t):
                        rows[b, r0 + k, sl] = rows[b, r0 + k, sl] + pvs[k]
                    return pvs

                lax.fori_loop(0, n_groups, grp, pvs)

        ga(0, 0).start()
        ga(1, 1).start()
        pos_cp.wait()

        for k in range(_NBUF):
            ga(k, k).wait()
            if k < 2:
                ga(k + 2, k + 2).start()
            else:
                wb(k - 2, k - 2).wait()
                ga(k + 2, k - 2).start()
            add_chunk(k)
            wb(k, k).start()

        def mid(p, _):
            c0 = p * _NBUF
            for k in range(_NBUF):
                c = c0 + k
                ga(c, k).wait()
                b2 = (k + 2) % _NBUF
                wb(c - 2, b2).wait()
                ga(c + 2, b2).start()
                add_chunk(k)
                wb(c, k).start()
            return 0

        lax.fori_loop(1, n_rounds - 1, mid, 0)

        cL = (n_rounds - 1) * _NBUF
        for k in range(_NBUF):
            c = cL + k
            ga(c, k).wait()
            if k < 2:
                b2 = (k + 2) % _NBUF
                wb(c - 2, b2).wait()
                ga(c + 2, b2).start()
            add_chunk(k)
            wb(c, k).start()

        for k in range(_NBUF):
            wb(cL + k, k).wait()

    return body(ids_3d, token_table, pos_table)


def kernel(input_ids, token_table, pos_table):
    bq, tq = input_ids.shape
    vocab, d = token_table.shape
    n = bq * tq
    chunk = 128
    assert n % (_NW * chunk) == 0 and chunk % tq == 0 and d % _LANES == 0
    n_chunks = n // (_NW * chunk)
    assert n_chunks % _NBUF == 0 and n_chunks // _NBUF >= 2
    ids_3d = input_ids.astype(jnp.int32).reshape(_NW, n_chunks, chunk)
    out = _sc_embed(ids_3d, token_table, pos_table,
                    n_chunks=n_chunks, chunk=chunk, t=tq, d=d)
    return out.reshape(bq, tq, d)
